# baseline (speedup 1.0000x reference)
import functools

import jax
import jax.numpy as jnp
from jax import lax
from jax.experimental import pallas as pl
from jax.experimental.pallas import tpu as pltpu
from jax.experimental.pallas import tpu_sc as plsc

B = 16384
NF = 26
D = 64
NC = 2
NS = 16
NW = NC * NS
NSPLIT = 4
BS = B // NSPLIT
TOT_S = BS * NF
PER_W = TOT_S // NW
CHUNK = 128
NCHUNK = PER_W // CHUNK
NBUF = 13
NSTEP = NCHUNK // NBUF

_mesh = plsc.VectorSubcoreMesh(
    core_axis_name="c", subcore_axis_name="s", num_cores=NC, num_subcores=NS
)


@functools.partial(
    pl.kernel,
    out_type=jax.ShapeDtypeStruct((TOT_S, D), jnp.float32),
    mesh=_mesh,
    scratch_types=[
        pltpu.VMEM((NCHUNK, CHUNK), jnp.int32),
        pltpu.VMEM((NBUF, CHUNK, D), jnp.float32),
        pltpu.SemaphoreType.DMA((NBUF,)),
        pltpu.SemaphoreType.DMA((NBUF,)),
    ],
    compiler_params=pltpu.CompilerParams(use_tc_tiling_on_sc=False),
)
def _gather_band(pe_hbm, idx_hbm, out_hbm, idx_v, rows_v, sem_in, sem_out):
    wid = lax.axis_index("s") * NC + lax.axis_index("c")
    base = wid * PER_W
    pltpu.sync_copy(idx_hbm.at[wid], idx_v)

    def gather_start(g, b):
        pltpu.async_copy(pe_hbm.at[idx_v.at[g]], rows_v.at[b], sem_in.at[b])

    def gather_wait(b):
        pltpu.make_async_copy(
            pe_hbm.at[idx_v.at[0]], rows_v.at[b], sem_in.at[b]
        ).wait()

    def wb_start(g, b):
        pltpu.async_copy(
            rows_v.at[b], out_hbm.at[pl.ds(base + g * CHUNK, CHUNK)], sem_out.at[b]
        )

    def wb_wait(b):
        pltpu.make_async_copy(
            rows_v.at[b], out_hbm.at[pl.ds(base, CHUNK)], sem_out.at[b]
        ).wait()

    for b in range(NBUF):
        gather_start(b, b)

    def step(j, carry):
        for b in range(NBUF):
            gather_wait(b)
            wb_start(j * NBUF + b, b)
        for b in range(NBUF):
            wb_wait(b)
            gather_start((j + 1) * NBUF + b, b)
        return carry

    lax.fori_loop(0, NSTEP - 1, step, 0)

    for b in range(NBUF):
        gather_wait(b)
        wb_start((NSTEP - 1) * NBUF + b, b)
    for b in range(NBUF):
        wb_wait(b)


def kernel(x, pe, dev=0):
    out = jnp.zeros((B, NF * D), jnp.float32)
    for k in range(NSPLIT):
        xk = lax.slice(x, (k * BS, 0), ((k + 1) * BS, NF))
        ok = _gather_band(pe, xk.reshape(NW, NCHUNK, CHUNK))
        out = lax.dynamic_update_slice(out, ok.reshape(BS, NF * D), (k * BS, 0))
    return out


# speedup vs baseline: 1.2450x; 1.2450x over previous
import functools

import jax
import jax.numpy as jnp
from jax import lax
from jax.experimental import pallas as pl
from jax.experimental.pallas import tpu as pltpu
from jax.experimental.pallas import tpu_sc as plsc

B = 16384
NF = 26
D = 64
XP = 128
TOT = B * NF
NC = 2
NS = 16
NW = NC * NS
ROWS_W = B // NW
PER_W = ROWS_W * NF
CHUNK = 128
NCHUNK = PER_W // CHUNK
NBUF = 4
NSTEP = NCHUNK // NBUF
L = 16

_mesh = plsc.VectorSubcoreMesh(
    core_axis_name="c", subcore_axis_name="s", num_cores=NC, num_subcores=NS
)


@functools.partial(
    pl.kernel,
    out_type=jax.ShapeDtypeStruct((TOT, D), jnp.float32),
    mesh=_mesh,
    scratch_types=[
        pltpu.VMEM((ROWS_W, XP), jnp.int32),
        pltpu.VMEM((PER_W + L,), jnp.int32),
        pltpu.VMEM((NBUF, CHUNK, D), jnp.float32),
        pltpu.SemaphoreType.DMA((NBUF,)),
        pltpu.SemaphoreType.DMA((NBUF,)),
    ],
    compiler_params=pltpu.CompilerParams(use_tc_tiling_on_sc=False),
)
def _gather_kernel(pe_hbm, xp_hbm, out_hbm, xraw_v, idx_v, rows_v, sem_in, sem_out):
    wid = lax.axis_index("s") * NC + lax.axis_index("c")
    base = wid * PER_W
    pltpu.sync_copy(xp_hbm.at[pl.ds(wid * ROWS_W, ROWS_W)], xraw_v)


    def compact_row(r, carry):
        lo = xraw_v[r, pl.ds(0, L)]
        hi = xraw_v[r, pl.ds(L, L)]
        idx_v[pl.ds(r * NF, L)] = lo
        idx_v[pl.ds(r * NF + L, L)] = hi
        return carry

    lax.fori_loop(0, ROWS_W, compact_row, 0)

    def gather_start(g, b):
        pltpu.async_copy(pe_hbm.at[idx_v.at[pl.ds(g * CHUNK, CHUNK)]], rows_v.at[b], sem_in.at[b])

    def gather_wait(b):
        pltpu.make_async_copy(
            pe_hbm.at[idx_v.at[pl.ds(0, CHUNK)]], rows_v.at[b], sem_in.at[b]
        ).wait()

    def wb_start(g, b):
        pltpu.async_copy(
            rows_v.at[b], out_hbm.at[pl.ds(base + g * CHUNK, CHUNK)], sem_out.at[b]
        )

    def wb_wait(b):
        pltpu.make_async_copy(
            rows_v.at[b], out_hbm.at[pl.ds(base, CHUNK)], sem_out.at[b]
        ).wait()

    for b in range(NBUF):
        gather_start(b, b)

    def step(j, carry):
        for b in range(NBUF):
            gather_wait(b)
            wb_start(j * NBUF + b, b)
        for b in range(NBUF):
            wb_wait(b)
            gather_start((j + 1) * NBUF + b, b)
        return carry

    lax.fori_loop(0, NSTEP - 1, step, 0)

    for b in range(NBUF):
        gather_wait(b)
        wb_start((NSTEP - 1) * NBUF + b, b)
    for b in range(NBUF):
        wb_wait(b)


def kernel(x, pe, dev=0):
    xp = jnp.pad(x, ((0, 0), (0, XP - NF)))
    out = _gather_kernel(pe, xp)
    return out.reshape(B, NF * D)


# speedup vs baseline: 1.2750x; 1.0241x over previous
import functools

import jax
import jax.numpy as jnp
from jax import lax
from jax.experimental import pallas as pl
from jax.experimental.pallas import tpu as pltpu
from jax.experimental.pallas import tpu_sc as plsc

B = 16384
NF = 26
D = 64
DP = 128
NC = 2
NS = 16
NW = NC * NS
ROWS_W = B // NW
RB = 8
CHUNK = RB * NF
NBLK = ROWS_W // RB
PER_W = ROWS_W * NF
QR = ROWS_W // 4
L = 16

_mesh = plsc.VectorSubcoreMesh(
    core_axis_name="c", subcore_axis_name="s", num_cores=NC, num_subcores=NS
)


@functools.partial(
    pl.kernel,
    out_type=jax.ShapeDtypeStruct((B, NF * D), jnp.float32),
    mesh=_mesh,
    scratch_types=[
        pltpu.VMEM((QR, NF), jnp.int32),
        pltpu.VMEM((PER_W + L,), jnp.int32),
        pltpu.VMEM((2, CHUNK, DP), jnp.float32),
        pltpu.VMEM((2, RB, NF * D), jnp.float32),
        pltpu.SemaphoreType.DMA((2,)),
        pltpu.SemaphoreType.DMA((2,)),
    ],
    compiler_params=pltpu.CompilerParams(use_tc_tiling_on_sc=True),
)
def _gather_kernel(pep_hbm, x_hbm, out_hbm, xq_v, idx_v, pair_v, wb_v,
                   sem_in, sem_out):
    wid = lax.axis_index("s") * NC + lax.axis_index("c")
    row0 = wid * ROWS_W

    def stage_quarter(q, carry):
        pltpu.sync_copy(x_hbm.at[pl.ds(row0 + q * QR, QR)], xq_v)

        def compact_row(r, carry2):
            lo = xq_v[r, pl.ds(0, L)]
            hi = xq_v[r, pl.ds(NF - L, L)]
            flat = (q * QR + r) * NF
            idx_v[pl.ds(flat, L)] = lo
            idx_v[pl.ds(flat + NF - L, L)] = hi
            return carry2

        lax.fori_loop(0, QR, compact_row, 0)
        return carry

    lax.fori_loop(0, 4, stage_quarter, 0)

    def gather_start(j, b):
        pltpu.async_copy(
            pep_hbm.at[idx_v.at[pl.ds(j * CHUNK, CHUNK)]], pair_v.at[b],
            sem_in.at[b],
        )

    def gather_wait(b):
        pltpu.make_async_copy(
            pep_hbm.at[idx_v.at[pl.ds(0, CHUNK)]], pair_v.at[b], sem_in.at[b]
        ).wait()

    def wb_start(j, b):
        pltpu.async_copy(
            wb_v.at[b], out_hbm.at[pl.ds(row0 + j * RB, RB)], sem_out.at[b]
        )

    def wb_wait(b):
        pltpu.make_async_copy(
            wb_v.at[b], out_hbm.at[pl.ds(row0, RB)], sem_out.at[b]
        ).wait()

    def compact_block(b):
        def do_row(r, carry):
            for f in range(NF):
                for k in range(0, D, L):
                    wb_v[b, r, pl.ds(f * D + k, L)] = (
                        pair_v[b, r * NF + f, pl.ds(k, L)]
                    )
            return carry

        lax.fori_loop(0, RB, do_row, 0)

    gather_start(0, 0)
    gather_start(1, 1)

    def step(j2, carry):
        for b in range(2):
            j = j2 * 2 + b
            gather_wait(b)

            @pl.when(j2 >= 1)
            def _():
                wb_wait(b)

            compact_block(b)
            wb_start(j, b)
            nxt = lax.select(j + 2 < NBLK, j + 2, 0)
            gather_start(nxt, b)
        return carry

    lax.fori_loop(0, NBLK // 2, step, 0)

    for b in range(2):
        gather_wait(b)
        wb_wait(b)


def kernel(x, pe, dev=0):
    pep = jnp.pad(pe, ((0, 0), (0, DP - D)))
    return _gather_kernel(pep, x)


# speedup vs baseline: 1.6482x; 1.2926x over previous
import functools

import jax
import jax.numpy as jnp
from jax import lax
from jax.experimental import pallas as pl
from jax.experimental.pallas import tpu as pltpu
from jax.experimental.pallas import tpu_sc as plsc

B = 16384
NF = 26
D = 64
DP = 128
NC = 2
NS = 16
NW = NC * NS
ROWS_W = B // NW
RB = 8
CHUNK = RB * NF
NBLK = ROWS_W // RB
PER_W = ROWS_W * NF
QR = ROWS_W // 4
L = 16

_mesh = plsc.VectorSubcoreMesh(
    core_axis_name="c", subcore_axis_name="s", num_cores=NC, num_subcores=NS
)


@functools.partial(
    pl.kernel,
    out_type=jax.ShapeDtypeStruct((B, NF * D), jnp.float32),
    mesh=_mesh,
    scratch_types=[
        pltpu.VMEM((QR, NF), jnp.int32),
        pltpu.VMEM((PER_W + L,), jnp.int32),
        pltpu.VMEM((2, CHUNK, DP), jnp.float32),
        pltpu.VMEM((2, RB, NF * D), jnp.float32),
        pltpu.SemaphoreType.DMA((2,)),
        pltpu.SemaphoreType.DMA((2,)),
    ],
    compiler_params=pltpu.CompilerParams(use_tc_tiling_on_sc=True),
)
def _gather_kernel(pep_hbm, x_hbm, out_hbm, xq_v, idx_v, pair_v, wb_v,
                   sem_in, sem_out):
    wid = lax.axis_index("s") * NC + lax.axis_index("c")
    row0 = wid * ROWS_W

    def stage_quarter(q, carry):
        pltpu.sync_copy(x_hbm.at[pl.ds(row0 + q * QR, QR)], xq_v)

        def compact_row(r, carry2):
            lo = xq_v[r, pl.ds(0, L)]
            hi = xq_v[r, pl.ds(NF - L, L)]
            flat = (q * QR + r) * NF
            idx_v[pl.ds(flat, L)] = lo
            idx_v[pl.ds(flat + NF - L, L)] = hi
            return carry2

        lax.fori_loop(0, QR, compact_row, 0)
        return carry

    lax.fori_loop(0, 4, stage_quarter, 0)

    def gather_start(j, b):
        pltpu.async_copy(
            pep_hbm.at[idx_v.at[pl.ds(j * CHUNK, CHUNK)]], pair_v.at[b],
            sem_in.at[b],
        )

    def gather_wait(b):
        pltpu.make_async_copy(
            pep_hbm.at[idx_v.at[pl.ds(0, CHUNK)]], pair_v.at[b], sem_in.at[b]
        ).wait()

    def wb_start(j, b):
        pltpu.async_copy(
            wb_v.at[b], out_hbm.at[pl.ds(row0 + j * RB, RB)], sem_out.at[b]
        )

    def wb_wait(b):
        pltpu.make_async_copy(
            wb_v.at[b], out_hbm.at[pl.ds(row0, RB)], sem_out.at[b]
        ).wait()

    def compact_block(b):
        for r in range(RB):
            for f in range(NF):
                for k in range(0, D, L):
                    wb_v[b, r, pl.ds(f * D + k, L)] = (
                        pair_v[b, r * NF + f, pl.ds(k, L)]
                    )

    gather_start(0, 0)
    gather_start(1, 1)

    def step(j2, carry):
        for b in range(2):
            j = j2 * 2 + b
            gather_wait(b)

            @pl.when(j2 >= 1)
            def _():
                wb_wait(b)

            compact_block(b)
            wb_start(j, b)
            nxt = lax.select(j + 2 < NBLK, j + 2, 0)
            gather_start(nxt, b)
        return carry

    lax.fori_loop(0, NBLK // 2, step, 0)

    for b in range(2):
        gather_wait(b)
        wb_wait(b)


def kernel(x, pe, dev=0):
    pep = jnp.pad(pe, ((0, 0), (0, DP - D)))
    return _gather_kernel(pep, x)


# speedup vs baseline: 1.6570x; 1.0054x over previous
import functools

import jax
import jax.numpy as jnp
from jax import lax
from jax.experimental import pallas as pl
from jax.experimental.pallas import tpu as pltpu
from jax.experimental.pallas import tpu_sc as plsc

B = 16384
NF = 26
D = 64
DP = 128
NC = 2
NS = 16
NW = NC * NS
ROWS_W = B // NW
RB = 8
CHUNK = RB * NF
NBLK = ROWS_W // RB
PER_W = ROWS_W * NF
QR = ROWS_W // 4
L = 16

_mesh = plsc.VectorSubcoreMesh(
    core_axis_name="c", subcore_axis_name="s", num_cores=NC, num_subcores=NS
)


@functools.partial(
    pl.kernel,
    out_type=jax.ShapeDtypeStruct((B, NF * D), jnp.float32),
    mesh=_mesh,
    scratch_types=[
        pltpu.VMEM((PER_W,), jnp.int32),
        pltpu.VMEM((2, CHUNK, DP), jnp.float32),
        pltpu.VMEM((2, RB, NF * D), jnp.float32),
        pltpu.SemaphoreType.DMA((2,)),
        pltpu.SemaphoreType.DMA((2,)),
    ],
    compiler_params=pltpu.CompilerParams(use_tc_tiling_on_sc=True),
)
def _gather_kernel(pep_hbm, xf_hbm, out_hbm, idx_v, pair_v, wb_v,
                   sem_in, sem_out):
    wid = lax.axis_index("s") * NC + lax.axis_index("c")
    row0 = wid * ROWS_W

    pltpu.sync_copy(xf_hbm.at[pl.ds(wid * PER_W, PER_W)], idx_v)

    def gather_start(j, b):
        pltpu.async_copy(
            pep_hbm.at[idx_v.at[pl.ds(j * CHUNK, CHUNK)]], pair_v.at[b],
            sem_in.at[b],
        )

    def gather_wait(b):
        pltpu.make_async_copy(
            pep_hbm.at[idx_v.at[pl.ds(0, CHUNK)]], pair_v.at[b], sem_in.at[b]
        ).wait()

    def wb_start(j, b):
        pltpu.async_copy(
            wb_v.at[b], out_hbm.at[pl.ds(row0 + j * RB, RB)], sem_out.at[b]
        )

    def wb_wait(b):
        pltpu.make_async_copy(
            wb_v.at[b], out_hbm.at[pl.ds(row0, RB)], sem_out.at[b]
        ).wait()

    def compact_block(b):
        for r in range(RB):
            for f in range(NF):
                for k in range(0, D, L):
                    wb_v[b, r, pl.ds(f * D + k, L)] = (
                        pair_v[b, r * NF + f, pl.ds(k, L)]
                    )

    gather_start(0, 0)
    gather_start(1, 1)

    def step(j2, carry):
        for b in range(2):
            j = j2 * 2 + b
            gather_wait(b)

            @pl.when(j2 >= 1)
            def _():
                wb_wait(b)

            compact_block(b)
            wb_start(j, b)
            nxt = lax.select(j + 2 < NBLK, j + 2, 0)
            gather_start(nxt, b)
        return carry

    lax.fori_loop(0, NBLK // 2, step, 0)

    for b in range(2):
        gather_wait(b)
        wb_wait(b)


def kernel(x, pe, dev=0):
    pep = jnp.pad(pe, ((0, 0), (0, DP - D)))
    return _gather_kernel(pep, x.reshape(B * NF))
